# baseline (speedup 1.0000x reference)
import functools

import jax
import jax.numpy as jnp
from jax import lax
from jax.experimental import pallas as pl
from jax.experimental.pallas import tpu as pltpu
from jax.experimental.pallas import tpu_sc as plsc


@functools.lru_cache(maxsize=None)
def _build_gather(V, D, B, S):
    info = plsc.get_sparse_core_info()
    NC, NS = info.num_cores, info.num_subcores
    NW = NC * NS
    N = B * S
    assert N % (8 * NW) == 0
    BPW = N // NW
    assert S % BPW == 0
    SPB = S // BPW
    C = 8
    NBUF = 12
    NCHUNK = BPW // C
    assert BPW % C == 0

    mesh = plsc.VectorSubcoreMesh(core_axis_name="c", subcore_axis_name="s")

    @functools.partial(
        pl.kernel,
        mesh=mesh,
        out_type=jax.ShapeDtypeStruct((N, D), jnp.float32),
        scratch_types=[
            pltpu.VMEM((BPW,), jnp.int32),
            pltpu.VMEM((NBUF, C, D), jnp.float32),
        ] + [pltpu.SemaphoreType.DMA] * (2 * NBUF),
    )
    def gather(table_hbm, ids_hbm, out_hbm, idx_v, rows_v, *sems):
        gsem = sems[:NBUF]
        ssem = sems[NBUF:]
        wid = lax.axis_index("s") * NC + lax.axis_index("c")
        base = wid * BPW
        pltpu.sync_copy(
            ids_hbm.at[wid // SPB, pl.ds((wid % SPB) * BPW, BPW)], idx_v)

        def start_gather(g):
            b = g % NBUF
            return pltpu.async_copy(
                table_hbm.at[idx_v.at[pl.ds(g * C, C)]], rows_v.at[b],
                gsem[b])

        AHEAD = NBUF - 1
        gathers = {}
        scatters = {}
        for g in range(min(AHEAD, NCHUNK)):
            gathers[g] = start_gather(g)
        for g in range(NCHUNK):
            b = g % NBUF
            nxt = g + AHEAD
            if nxt < NCHUNK:
                if nxt >= NBUF:
                    scatters[nxt - NBUF].wait()
                gathers[nxt] = start_gather(nxt)
            gathers[g].wait()
            scatters[g] = pltpu.async_copy(
                rows_v.at[b], out_hbm.at[pl.ds(base + g * C, C)], ssem[b])
        for g in range(max(0, NCHUNK - NBUF), NCHUNK):
            scatters[g].wait()

    return gather


def kernel(input_ids, embed_table):
    B, S = input_ids.shape
    V, D = embed_table.shape
    ids = input_ids.astype(jnp.int32)
    out = _build_gather(V, D, B, S)(embed_table, ids)
    return out.reshape(B, S, D)


# speedup vs baseline: 1.0280x; 1.0280x over previous
import functools

import jax
import jax.numpy as jnp
from jax import lax
from jax.experimental import pallas as pl
from jax.experimental.pallas import tpu as pltpu
from jax.experimental.pallas import tpu_sc as plsc


@functools.lru_cache(maxsize=None)
def _build_gather(V, D, B, S):
    info = plsc.get_sparse_core_info()
    NC, NS = info.num_cores, info.num_subcores
    NW = NC * NS
    N = B * S
    assert N % (8 * NW) == 0
    BPW = N // NW
    assert S % BPW == 0
    SPB = S // BPW
    C = 16
    NBUF = 7
    NCHUNK = BPW // C
    assert BPW % C == 0

    mesh = plsc.VectorSubcoreMesh(core_axis_name="c", subcore_axis_name="s")

    @functools.partial(
        pl.kernel,
        mesh=mesh,
        out_type=jax.ShapeDtypeStruct((N, D), jnp.float32),
        scratch_types=[
            pltpu.VMEM((BPW,), jnp.int32),
            pltpu.VMEM((NBUF, C, D), jnp.float32),
        ] + [pltpu.SemaphoreType.DMA] * (2 * NBUF),
    )
    def gather(table_hbm, ids_hbm, out_hbm, idx_v, rows_v, *sems):
        gsem = sems[:NBUF]
        ssem = sems[NBUF:]
        wid = lax.axis_index("s") * NC + lax.axis_index("c")
        base = wid * BPW
        pltpu.sync_copy(
            ids_hbm.at[wid // SPB, pl.ds((wid % SPB) * BPW, BPW)], idx_v)

        def start_gather(g):
            b = g % NBUF
            return pltpu.async_copy(
                table_hbm.at[idx_v.at[pl.ds(g * C, C)]], rows_v.at[b],
                gsem[b])

        AHEAD = NBUF - 1
        gathers = {}
        scatters = {}
        for g in range(min(AHEAD, NCHUNK)):
            gathers[g] = start_gather(g)
        for g in range(NCHUNK):
            b = g % NBUF
            nxt = g + AHEAD
            if nxt < NCHUNK:
                if nxt >= NBUF:
                    scatters[nxt - NBUF].wait()
                gathers[nxt] = start_gather(nxt)
            gathers[g].wait()
            scatters[g] = pltpu.async_copy(
                rows_v.at[b], out_hbm.at[pl.ds(base + g * C, C)], ssem[b])
        for g in range(max(0, NCHUNK - NBUF), NCHUNK):
            scatters[g].wait()

    return gather


def kernel(input_ids, embed_table):
    B, S = input_ids.shape
    V, D = embed_table.shape
    ids = input_ids.astype(jnp.int32)
    out = _build_gather(V, D, B, S)(embed_table, ids)
    return out.reshape(B, S, D)


# speedup vs baseline: 1.0320x; 1.0039x over previous
import functools

import jax
import jax.numpy as jnp
from jax import lax
from jax.experimental import pallas as pl
from jax.experimental.pallas import tpu as pltpu
from jax.experimental.pallas import tpu_sc as plsc


@functools.lru_cache(maxsize=None)
def _build_gather(V, D, B, S):
    info = plsc.get_sparse_core_info()
    NC, NS = info.num_cores, info.num_subcores
    NW = NC * NS
    N = B * S
    assert N % (8 * NW) == 0
    BPW = N // NW
    assert S % BPW == 0
    SPB = S // BPW
    C = 16
    NBUF = 6
    NCHUNK = BPW // C
    assert BPW % C == 0

    mesh = plsc.VectorSubcoreMesh(core_axis_name="c", subcore_axis_name="s")

    @functools.partial(
        pl.kernel,
        mesh=mesh,
        out_type=jax.ShapeDtypeStruct((N, D), jnp.float32),
        scratch_types=[
            pltpu.VMEM((BPW,), jnp.int32),
            pltpu.VMEM((NBUF, C, D), jnp.float32),
        ] + [pltpu.SemaphoreType.DMA] * (2 * NBUF),
    )
    def gather(table_hbm, ids_hbm, out_hbm, idx_v, rows_v, *sems):
        gsem = sems[:NBUF]
        ssem = sems[NBUF:]
        wid = lax.axis_index("s") * NC + lax.axis_index("c")
        base = wid * BPW
        pltpu.sync_copy(
            ids_hbm.at[wid // SPB, pl.ds((wid % SPB) * BPW, BPW)], idx_v)

        def start_gather(g):
            b = g % NBUF
            return pltpu.async_copy(
                table_hbm.at[idx_v.at[pl.ds(g * C, C)]], rows_v.at[b],
                gsem[b])

        AHEAD = NBUF - 1
        gathers = {}
        scatters = {}
        for g in range(min(AHEAD, NCHUNK)):
            gathers[g] = start_gather(g)
        for g in range(NCHUNK):
            b = g % NBUF
            gathers[g].wait()
            scatters[g] = pltpu.async_copy(
                rows_v.at[b], out_hbm.at[pl.ds(base + g * C, C)], ssem[b])
            nxt = g + AHEAD
            if nxt < NCHUNK:
                if nxt >= NBUF:
                    scatters[nxt - NBUF].wait()
                gathers[nxt] = start_gather(nxt)
        for g in range(max(0, NCHUNK - NBUF), NCHUNK):
            scatters[g].wait()

    return gather


def kernel(input_ids, embed_table):
    B, S = input_ids.shape
    V, D = embed_table.shape
    ids = input_ids.astype(jnp.int32)
    out = _build_gather(V, D, B, S)(embed_table, ids)
    return out.reshape(B, S, D)


# speedup vs baseline: 1.0384x; 1.0063x over previous
import functools

import jax
import jax.numpy as jnp
from jax import lax
from jax.experimental import pallas as pl
from jax.experimental.pallas import tpu as pltpu
from jax.experimental.pallas import tpu_sc as plsc


@functools.lru_cache(maxsize=None)
def _build_gather(V, D, B, S):
    info = plsc.get_sparse_core_info()
    NC, NS = info.num_cores, info.num_subcores
    NW = NC * NS
    N = B * S
    assert N % (8 * NW) == 0
    BPW = N // NW
    assert S % BPW == 0
    SPB = S // BPW
    C = 16
    NBUF = 6
    NCHUNK = BPW // C
    assert BPW % C == 0

    mesh = plsc.VectorSubcoreMesh(core_axis_name="c", subcore_axis_name="s")

    @functools.partial(
        pl.kernel,
        mesh=mesh,
        out_type=jax.ShapeDtypeStruct((N, D), jnp.float32),
        scratch_types=[
            pltpu.VMEM((BPW,), jnp.int32),
            pltpu.VMEM((NBUF, C, D), jnp.float32),
        ] + [pltpu.SemaphoreType.DMA] * (2 * NBUF),
    )
    def gather(table_hbm, ids_hbm, out_hbm, idx_v, rows_v, *sems):
        gsem = sems[:NBUF]
        ssem = sems[NBUF:]
        wid = lax.axis_index("s") * NC + lax.axis_index("c")
        base = wid * BPW
        row = wid // SPB
        col = (wid % SPB) * BPW

        def start_gather(g):
            b = g % NBUF
            return pltpu.async_copy(
                table_hbm.at[idx_v.at[pl.ds(g * C, C)]], rows_v.at[b],
                gsem[b])

        AHEAD = NBUF - 1
        gathers = {}
        scatters = {}
        pltpu.sync_copy(ids_hbm.at[row, pl.ds(col, BPW)], idx_v)
        for g in range(min(AHEAD, NCHUNK)):
            gathers[g] = start_gather(g)
        for g in range(NCHUNK):
            b = g % NBUF
            nxt = g + AHEAD
            if nxt < NCHUNK:
                if nxt >= NBUF:
                    scatters[nxt - NBUF].wait()
                gathers[nxt] = start_gather(nxt)
            gathers[g].wait()
            scatters[g] = pltpu.async_copy(
                rows_v.at[b], out_hbm.at[pl.ds(base + g * C, C)], ssem[b])
        for g in range(max(0, NCHUNK - NBUF), NCHUNK):
            scatters[g].wait()

    return gather


def kernel(input_ids, embed_table):
    B, S = input_ids.shape
    V, D = embed_table.shape
    ids = input_ids.astype(jnp.int32)
    out = _build_gather(V, D, B, S)(embed_table, ids)
    return out.reshape(B, S, D)
